# baseline (speedup 1.0000x reference)
import functools

import jax
import jax.numpy as jnp
from jax import lax
from jax.experimental import pallas as pl
from jax.experimental.pallas import tpu as pltpu
from jax.experimental.pallas import tpu_sc as plsc

_MAXV = 100000
_D = 64
_P = 112
_PP = 113


@functools.partial(jax.jit, static_argnums=(2, 3))
def _bow_gather(idx, table, B, HW):
    info = plsc.get_sparse_core_info()
    NC, NS = info.num_cores, info.num_subcores
    NW = NC * NS
    bpw = B // NW
    cpb = HW // _P
    nchunks = bpw * cpb

    mesh = plsc.VectorSubcoreMesh(core_axis_name="c", subcore_axis_name="s")

    @functools.partial(
        pl.kernel,
        mesh=mesh,
        compiler_params=pltpu.CompilerParams(
            use_tc_tiling_on_sc=False, needs_layout_passes=False
        ),
        out_type=jax.ShapeDtypeStruct((B, _D, HW), jnp.float32),
        scratch_types=[
            pltpu.VMEM((bpw * 3 * HW,), jnp.int32),
            pltpu.VMEM((2, 3, _P, _D), jnp.float32),
            pltpu.VMEM((2, _D, _PP), jnp.float32),
            (pltpu.SemaphoreType.DMA,) * 2,
            (pltpu.SemaphoreType.DMA,) * 2,
        ],
    )
    def k(idx_hbm, table_hbm, out_hbm, idx_v, rows_v, out_v, gsems, osems):
        wid = lax.axis_index("s") * NC + lax.axis_index("c")
        iota = lax.iota(jnp.int32, 16)

        pltpu.sync_copy(idx_hbm.at[pl.ds(wid * bpw * 3 * HW, bpw * 3 * HW)], idx_v)

        def chunk_off(t):
            bl = t // cpb
            off = (t % cpb) * _P
            return bl * 3 * HW + off, bl, off

        def fire_gathers(t, slot):
            base, _, _ = chunk_off(t)
            for c in range(3):
                pltpu.async_copy(
                    table_hbm.at[idx_v.at[pl.ds(base + c * HW, _P)]],
                    rows_v.at[slot, c],
                    gsems[slot],
                )

        def wait_gathers(slot):
            for c in range(3):
                pltpu.make_async_copy(
                    table_hbm.at[idx_v.at[pl.ds(0, _P)]],
                    rows_v.at[slot, c],
                    gsems[slot],
                ).wait()

        def out_dma(t, slot):
            _, bl, off = chunk_off(t)
            b = wid * bpw + bl
            return pltpu.make_async_copy(
                out_v.at[slot, :, pl.ds(0, _P)],
                out_hbm.at[b, :, pl.ds(off, _P)],
                osems[slot],
            )

        fire_gathers(0, 0)
        fire_gathers(1, 1)

        dvecs = [iota + gd * 16 for gd in range(_D // 16)]

        def step(j, _):
            for slot in (0, 1):
                t = 2 * j + slot
                wait_gathers(slot)

                @pl.when(j > 0)
                def _():
                    out_dma(t - 2, slot).wait()

                @plsc.parallel_loop(0, _P, 1, unroll=8)
                def p_body(p):
                    psel = jnp.full((16,), p, jnp.int32)
                    for gd in range(_D // 16):
                        sl = pl.ds(gd * 16, 16)
                        v = (
                            rows_v[slot, 0, p, sl]
                            + rows_v[slot, 1, p, sl]
                            + rows_v[slot, 2, p, sl]
                        )
                        plsc.store_scatter(
                            out_v.at[slot], [dvecs[gd], psel], v
                        )
                out_dma(t, slot).start()

                @pl.when(t + 2 < nchunks)
                def _():
                    fire_gathers(t + 2, slot)
            return 0

        lax.fori_loop(0, nchunks // 2, step, 0)
        out_dma(nchunks - 2, 0).wait()
        out_dma(nchunks - 1, 1).wait()

    return k(idx, table)


def kernel(inputs, table):
    B, C, H, W = inputs.shape
    HW = H * W
    offsets = jnp.arange(C, dtype=jnp.int32) * _MAXV
    idx = (inputs + offsets[None, :, None, None]).reshape(B * C * HW)
    out = _bow_gather(idx, table, B, HW)
    return out.reshape(B, _D, H, W)


# speedup vs baseline: 1.0043x; 1.0043x over previous
import functools

import jax
import jax.numpy as jnp
from jax import lax
from jax.experimental import pallas as pl
from jax.experimental.pallas import tpu as pltpu
from jax.experimental.pallas import tpu_sc as plsc

_MAXV = 100000
_D = 64
_P = 112
_PP = 113


@functools.partial(jax.jit, static_argnums=(2, 3))
def _bow_gather(idx, table, B, HW):
    info = plsc.get_sparse_core_info()
    NC, NS = info.num_cores, info.num_subcores
    NW = NC * NS
    bpw = B // NW
    cpb = HW // _P
    nchunks = bpw * cpb

    mesh = plsc.VectorSubcoreMesh(core_axis_name="c", subcore_axis_name="s")

    @functools.partial(
        pl.kernel,
        mesh=mesh,
        compiler_params=pltpu.CompilerParams(
            use_tc_tiling_on_sc=False, needs_layout_passes=False
        ),
        out_type=jax.ShapeDtypeStruct((B, _D, HW), jnp.float32),
        scratch_types=[
            pltpu.VMEM((bpw * 3 * HW,), jnp.int32),
            pltpu.VMEM((2, 3, _P, _D), jnp.float32),
            pltpu.VMEM((2, _D, _PP), jnp.float32),
            (pltpu.SemaphoreType.DMA,) * 2,
            (pltpu.SemaphoreType.DMA,) * 2,
        ],
    )
    def k(idx_hbm, table_hbm, out_hbm, idx_v, rows_v, out_v, gsems, osems):
        wid = lax.axis_index("s") * NC + lax.axis_index("c")
        iota = lax.iota(jnp.int32, 16)

        pltpu.sync_copy(idx_hbm.at[pl.ds(wid * bpw * 3 * HW, bpw * 3 * HW)], idx_v)

        def chunk_off(t):
            bl = t // cpb
            off = (t % cpb) * _P
            return bl * 3 * HW + off, bl, off

        def fire_gathers(t, slot):
            base, _, _ = chunk_off(t)
            for c in range(3):
                pltpu.async_copy(
                    table_hbm.at[idx_v.at[pl.ds(base + c * HW, _P)]],
                    rows_v.at[slot, c],
                    gsems[slot],
                )

        def wait_gathers(slot):
            for c in range(3):
                pltpu.make_async_copy(
                    table_hbm.at[idx_v.at[pl.ds(0, _P)]],
                    rows_v.at[slot, c],
                    gsems[slot],
                ).wait()

        def out_dma(t, slot):
            _, bl, off = chunk_off(t)
            b = wid * bpw + bl
            return pltpu.make_async_copy(
                out_v.at[slot, :, pl.ds(0, _P)],
                out_hbm.at[b, :, pl.ds(off, _P)],
                osems[slot],
            )

        fire_gathers(0, 0)
        fire_gathers(1, 1)

        dvecs = [iota + gd * 16 for gd in range(_D // 16)]

        def step(j, _):
            for slot in (0, 1):
                t = 2 * j + slot
                wait_gathers(slot)

                @pl.when(j > 0)
                def _():
                    out_dma(t - 2, slot).wait()

                @plsc.parallel_loop(0, _P, 1, unroll=4)
                def p_body(p):
                    psel = jnp.full((16,), p, jnp.int32)
                    for gd in range(_D // 16):
                        sl = pl.ds(gd * 16, 16)
                        v = (
                            rows_v[slot, 0, p, sl]
                            + rows_v[slot, 1, p, sl]
                            + rows_v[slot, 2, p, sl]
                        )
                        plsc.store_scatter(
                            out_v.at[slot], [dvecs[gd], psel], v
                        )
                out_dma(t, slot).start()

                @pl.when(t + 2 < nchunks)
                def _():
                    fire_gathers(t + 2, slot)
            return 0

        lax.fori_loop(0, nchunks // 2, step, 0)
        out_dma(nchunks - 2, 0).wait()
        out_dma(nchunks - 1, 1).wait()

    return k(idx, table)


def kernel(inputs, table):
    B, C, H, W = inputs.shape
    HW = H * W
    offsets = jnp.arange(C, dtype=jnp.int32) * _MAXV
    idx = (inputs + offsets[None, :, None, None]).reshape(B * C * HW)
    out = _bow_gather(idx, table, B, HW)
    return out.reshape(B, _D, H, W)


# speedup vs baseline: 1.0200x; 1.0156x over previous
import functools

import jax
import jax.numpy as jnp
from jax import lax
from jax.experimental import pallas as pl
from jax.experimental.pallas import tpu as pltpu
from jax.experimental.pallas import tpu_sc as plsc

_MAXV = 100000
_D = 64
_P = 112
_PP = 113


@functools.partial(jax.jit, static_argnums=(2, 3))
def _bow_gather(idx, table, B, HW):
    info = plsc.get_sparse_core_info()
    NC, NS = info.num_cores, info.num_subcores
    NW = NC * NS
    bpw = B // NW
    cpb = HW // _P
    nchunks = bpw * cpb

    mesh = plsc.VectorSubcoreMesh(core_axis_name="c", subcore_axis_name="s")

    @functools.partial(
        pl.kernel,
        mesh=mesh,
        compiler_params=pltpu.CompilerParams(
            use_tc_tiling_on_sc=False, needs_layout_passes=False
        ),
        out_type=jax.ShapeDtypeStruct((B, _D, HW), jnp.float32),
        scratch_types=[
            pltpu.VMEM((bpw * 3 * HW,), jnp.int32),
            pltpu.VMEM((3, 3, _P, _D), jnp.float32),
            pltpu.VMEM((3, _D, _PP), jnp.float32),
            (pltpu.SemaphoreType.DMA,) * 3,
            (pltpu.SemaphoreType.DMA,) * 3,
        ],
    )
    def k(idx_hbm, table_hbm, out_hbm, idx_v, rows_v, out_v, gsems, osems):
        wid = lax.axis_index("s") * NC + lax.axis_index("c")
        iota = lax.iota(jnp.int32, 16)

        pltpu.sync_copy(idx_hbm.at[pl.ds(wid * bpw * 3 * HW, bpw * 3 * HW)], idx_v)

        def chunk_off(t):
            bl = t // cpb
            off = (t % cpb) * _P
            return bl * 3 * HW + off, bl, off

        def fire_gathers(t, slot):
            base, _, _ = chunk_off(t)
            for c in range(3):
                pltpu.async_copy(
                    table_hbm.at[idx_v.at[pl.ds(base + c * HW, _P)]],
                    rows_v.at[slot, c],
                    gsems[slot],
                )

        def wait_gathers(slot):
            for c in range(3):
                pltpu.make_async_copy(
                    table_hbm.at[idx_v.at[pl.ds(0, _P)]],
                    rows_v.at[slot, c],
                    gsems[slot],
                ).wait()

        def out_dma(t, slot):
            _, bl, off = chunk_off(t)
            b = wid * bpw + bl
            return pltpu.make_async_copy(
                out_v.at[slot, :, pl.ds(0, _P)],
                out_hbm.at[b, :, pl.ds(off, _P)],
                osems[slot],
            )

        fire_gathers(0, 0)
        fire_gathers(1, 1)
        fire_gathers(2, 2)

        dvecs = [iota + gd * 16 for gd in range(_D // 16)]

        def step(j, _):
            for slot in (0, 1, 2):
                t = 3 * j + slot
                wait_gathers(slot)

                @pl.when(t >= 3)
                def _():
                    out_dma(t - 3, slot).wait()

                @plsc.parallel_loop(0, _P, 1, unroll=4)
                def p_body(p):
                    psel = jnp.full((16,), p, jnp.int32)
                    for gd in range(_D // 16):
                        sl = pl.ds(gd * 16, 16)
                        v = (
                            rows_v[slot, 0, p, sl]
                            + rows_v[slot, 1, p, sl]
                            + rows_v[slot, 2, p, sl]
                        )
                        plsc.store_scatter(
                            out_v.at[slot], [dvecs[gd], psel], v
                        )
                out_dma(t, slot).start()

                @pl.when(t + 3 < nchunks)
                def _():
                    fire_gathers(t + 3, slot)
            return 0

        nfull = nchunks // 3
        lax.fori_loop(0, nfull, step, 0)
        for t in range(3 * nfull, nchunks):
            slot = t % 3
            wait_gathers(slot)
            out_dma(t - 3, slot).wait()

            @plsc.parallel_loop(0, _P, 1, unroll=4)
            def p_body(p, slot=slot):
                psel = jnp.full((16,), p, jnp.int32)
                for gd in range(_D // 16):
                    sl = pl.ds(gd * 16, 16)
                    v = (
                        rows_v[slot, 0, p, sl]
                        + rows_v[slot, 1, p, sl]
                        + rows_v[slot, 2, p, sl]
                    )
                    plsc.store_scatter(out_v.at[slot], [dvecs[gd], psel], v)

            out_dma(t, slot).start()
        for t in range(nchunks - 3, nchunks):
            out_dma(t, t % 3).wait()

    return k(idx, table)


def kernel(inputs, table):
    B, C, H, W = inputs.shape
    HW = H * W
    offsets = jnp.arange(C, dtype=jnp.int32) * _MAXV
    idx = (inputs + offsets[None, :, None, None]).reshape(B * C * HW)
    out = _bow_gather(idx, table, B, HW)
    return out.reshape(B, _D, H, W)
